# Initial kernel scaffold; baseline (speedup 1.0000x reference)
import jax
import jax.numpy as jnp
from jax.experimental import pallas as pl


def kernel(xyz, points, W_center, b_center, W_dir, b_dir, gamma1, beta1, W_dir2, b_dir2, gamma2, beta2):
    raise NotImplementedError("write your pallas kernel here")


# speedup vs baseline: 28.6873x; 28.6873x over previous
import functools

import jax
import jax.numpy as jnp
from jax.experimental import pallas as pl

RADIUS = 0.2
DECAY_RADIUS = 0.4


def _prologue_kernel(pts_ref, wct_ref, bc_ref, wst_ref, bs_ref, ctr_ref, dir_ref):
    pts = pts_ref[...]
    ctr_ref[...] = (
        jnp.dot(pts, wct_ref[...], preferred_element_type=jnp.float32) + bc_ref[...]
    )
    dir_ref[...] = (
        jnp.dot(pts, wst_ref[...], preferred_element_type=jnp.float32) + bs_ref[...]
    )


def _agg_kernel(xyzr_ref, xyzbr_ref, xyzt_ref, xyzbt_ref, dir_ref, out_ref, *, ti, tj, d):
    n = xyzt_ref.shape[-1]
    r2c = RADIUS * RADIUS
    dr2 = DECAY_RADIUS * DECAY_RADIUS

    mm = jnp.dot(xyzbr_ref[0], xyzbt_ref[0], preferred_element_type=jnp.float32)
    xr = xyzr_ref[0]
    rn = (xr[:, 0:1] * xr[:, 0:1] + xr[:, 1:2] * xr[:, 1:2]) + xr[:, 2:3] * xr[:, 2:3]
    cx = xyzt_ref[0, 0:1, :]
    cy = xyzt_ref[0, 1:2, :]
    cz = xyzt_ref[0, 2:3, :]
    cn = (cx * cx + cy * cy) + cz * cz
    dist = -2.0 * mm
    dist = dist + rn
    dist = dist + cn
    m = jnp.min(dist, axis=1, keepdims=True)
    iota = jax.lax.broadcasted_iota(jnp.int32, (ti, n), 1)
    idx = jnp.min(jnp.where(dist == m, iota, n), axis=1, keepdims=True)
    onehot = (iota == idx).astype(jnp.float32)
    ctr = jax.lax.dot_general(
        onehot, xyzt_ref[0], (((1,), (1,)), ((), ())),
        preferred_element_type=jnp.float32, precision=jax.lax.Precision.HIGHEST,
    )
    xix = ctr[:, 0:1]
    xiy = ctr[:, 1:2]
    xiz = ctr[:, 2:3]

    acc = jnp.zeros((ti, d), jnp.float32)
    z = jnp.zeros((ti, 1), jnp.float32)
    for j0 in range(0, n, tj):
        xjx = xyzt_ref[0, 0:1, j0 : j0 + tj]
        xjy = xyzt_ref[0, 1:2, j0 : j0 + tj]
        xjz = xyzt_ref[0, 2:3, j0 : j0 + tj]
        dx = xjx - xix
        dy = xjy - xiy
        dz = xjz - xiz
        r2 = dx * dx + dy * dy + dz * dz
        rn_ = jnp.sqrt(r2)
        inv = 1.0 / (rn_ + 1e-8)
        dw = 1.0 - (r2 - r2c) / (dr2 - r2c)
        dw = jnp.maximum(dw, 0.0)
        z = z + jnp.sum(dw, axis=1, keepdims=True)
        q = dw * (inv * inv)
        dxb = dx.astype(jnp.bfloat16).astype(jnp.float32)
        dyb = dy.astype(jnp.bfloat16).astype(jnp.float32)
        dzb = dz.astype(jnp.bfloat16).astype(jnp.float32)
        for a, (c, pos) in enumerate(
            [(dzb, True), (dzb, False), (dyb, True), (dyb, False), (dxb, True), (dxb, False)]
        ):
            if pos:
                w = jnp.where(c > 0.0, c * c, 0.0)
            else:
                w = jnp.where(c < 0.0, c * c, 0.0)
            acc = acc + jnp.dot(
                w * q,
                dir_ref[0, j0 : j0 + tj, a * d : (a + 1) * d],
                preferred_element_type=jnp.float32, precision=jax.lax.Precision.HIGHEST,
            )
    out_ref[0] = acc / z


def _stats_kernel(x_ref, s_ref):
    x = x_ref[...]
    part = jnp.concatenate(
        [jnp.sum(x, axis=0, keepdims=True), jnp.sum(x * x, axis=0, keepdims=True)], 0
    )

    @pl.when(pl.program_id(0) == 0)
    def _():
        s_ref[...] = part

    @pl.when(pl.program_id(0) != 0)
    def _():
        s_ref[...] += part


def _mid_kernel(
    agg_ref, s1_ref, ctr_ref, g1_ref, b1_ref, w2t_ref, b2_ref, outp_ref, s2_ref, *, nt
):
    s = s1_ref[...]
    m1 = s[0:1] * (1.0 / nt)
    v1 = s[1:2] * (1.0 / nt) - m1 * m1
    dn = (agg_ref[...] - m1) / jnp.sqrt(v1 + 1e-5) * g1_ref[...] + b1_ref[...]
    dn = jnp.maximum(dn, 0.0)
    outp = (
        ctr_ref[...]
        + jnp.dot(dn.astype(jnp.bfloat16), w2t_ref[...],
                  preferred_element_type=jnp.float32)
        + b2_ref[...]
    )
    outp_ref[...] = outp
    part = jnp.concatenate(
        [jnp.sum(outp, axis=0, keepdims=True),
         jnp.sum(outp * outp, axis=0, keepdims=True)], 0
    )

    @pl.when(pl.program_id(0) == 0)
    def _():
        s2_ref[...] = part

    @pl.when(pl.program_id(0) != 0)
    def _():
        s2_ref[...] += part


def _final_kernel(outp_ref, s2_ref, g2_ref, bt2_ref, out_ref, *, nt):
    s = s2_ref[...]
    m2 = s[0:1] * (1.0 / nt)
    v2 = s[1:2] * (1.0 / nt) - m2 * m2
    o = (outp_ref[...] - m2) / jnp.sqrt(v2 + 1e-5) * g2_ref[...] + bt2_ref[...]
    out_ref[...] = jnp.maximum(o, 0.0)


def kernel(
    xyz,
    points,
    W_center,
    b_center,
    W_dir,
    b_dir,
    gamma1,
    beta1,
    W_dir2,
    b_dir2,
    gamma2,
    beta2,
):
    B, N, _ = xyz.shape
    C_IN = points.shape[-1]
    C_OUT = W_center.shape[0]
    D6 = W_dir.shape[0]
    D = D6 // 6

    pts_b = points.reshape(B * N, C_IN).astype(jnp.bfloat16)
    wst = W_dir.reshape(D, 6, C_IN).transpose(1, 0, 2).reshape(D6, C_IN).T
    bst = b_dir.reshape(D, 6).T.reshape(1, D6)

    TR = min(2048, B * N)
    center, dir_stack = pl.pallas_call(
        _prologue_kernel,
        grid=(B * N // TR,),
        in_specs=[
            pl.BlockSpec((TR, C_IN), lambda i: (i, 0)),
            pl.BlockSpec((C_IN, C_OUT), lambda i: (0, 0)),
            pl.BlockSpec((1, C_OUT), lambda i: (0, 0)),
            pl.BlockSpec((C_IN, D6), lambda i: (0, 0)),
            pl.BlockSpec((1, D6), lambda i: (0, 0)),
        ],
        out_specs=[
            pl.BlockSpec((TR, C_OUT), lambda i: (i, 0)),
            pl.BlockSpec((TR, D6), lambda i: (i, 0)),
        ],
        out_shape=[
            jax.ShapeDtypeStruct((B * N, C_OUT), jnp.float32),
            jax.ShapeDtypeStruct((B * N, D6), jnp.float32),
        ],
    )(pts_b, W_center.T.astype(jnp.bfloat16), b_center.reshape(1, C_OUT),
      wst.astype(jnp.bfloat16), bst)

    dir_b = dir_stack.reshape(B, N, D6)
    xyz_t = xyz.transpose(0, 2, 1)
    xyz_b16 = xyz.astype(jnp.bfloat16)
    xyz_t_b16 = xyz_t.astype(jnp.bfloat16)

    TI = min(256, N)
    TJ = min(512, N)
    agg = pl.pallas_call(
        functools.partial(_agg_kernel, ti=TI, tj=TJ, d=D),
        grid=(B, N // TI),
        in_specs=[
            pl.BlockSpec((1, TI, 3), lambda b, i: (b, i, 0)),
            pl.BlockSpec((1, TI, 3), lambda b, i: (b, i, 0)),
            pl.BlockSpec((1, 3, N), lambda b, i: (b, 0, 0)),
            pl.BlockSpec((1, 3, N), lambda b, i: (b, 0, 0)),
            pl.BlockSpec((1, N, D6), lambda b, i: (b, 0, 0)),
        ],
        out_specs=pl.BlockSpec((1, TI, D), lambda b, i: (b, i, 0)),
        out_shape=jax.ShapeDtypeStruct((B, N, D), jnp.float32),
    )(xyz, xyz_b16, xyz_t, xyz_t_b16, dir_b)

    agg_flat = agg.reshape(B * N, D)
    NT = B * N
    TE = min(2048, NT)
    GE = NT // TE
    s1 = pl.pallas_call(
        _stats_kernel,
        grid=(GE,),
        in_specs=[pl.BlockSpec((TE, D), lambda i: (i, 0))],
        out_specs=pl.BlockSpec((2, D), lambda i: (0, 0)),
        out_shape=jax.ShapeDtypeStruct((2, D), jnp.float32),
    )(agg_flat)

    outp, s2 = pl.pallas_call(
        functools.partial(_mid_kernel, nt=NT),
        grid=(GE,),
        in_specs=[
            pl.BlockSpec((TE, D), lambda i: (i, 0)),
            pl.BlockSpec((2, D), lambda i: (0, 0)),
            pl.BlockSpec((TE, C_OUT), lambda i: (i, 0)),
            pl.BlockSpec((1, D), lambda i: (0, 0)),
            pl.BlockSpec((1, D), lambda i: (0, 0)),
            pl.BlockSpec((D, C_OUT), lambda i: (0, 0)),
            pl.BlockSpec((1, C_OUT), lambda i: (0, 0)),
        ],
        out_specs=[
            pl.BlockSpec((TE, C_OUT), lambda i: (i, 0)),
            pl.BlockSpec((2, C_OUT), lambda i: (0, 0)),
        ],
        out_shape=[
            jax.ShapeDtypeStruct((NT, C_OUT), jnp.float32),
            jax.ShapeDtypeStruct((2, C_OUT), jnp.float32),
        ],
    )(agg_flat, s1, center, gamma1.reshape(1, D), beta1.reshape(1, D),
      W_dir2.T.astype(jnp.bfloat16), b_dir2.reshape(1, C_OUT))

    out = pl.pallas_call(
        functools.partial(_final_kernel, nt=NT),
        grid=(GE,),
        in_specs=[
            pl.BlockSpec((TE, C_OUT), lambda i: (i, 0)),
            pl.BlockSpec((2, C_OUT), lambda i: (0, 0)),
            pl.BlockSpec((1, C_OUT), lambda i: (0, 0)),
            pl.BlockSpec((1, C_OUT), lambda i: (0, 0)),
        ],
        out_specs=pl.BlockSpec((TE, C_OUT), lambda i: (i, 0)),
        out_shape=jax.ShapeDtypeStruct((NT, C_OUT), jnp.float32),
    )(outp, s2, gamma2.reshape(1, C_OUT), beta2.reshape(1, C_OUT))
    return out.reshape(B, N, C_OUT)


# speedup vs baseline: 52.6248x; 1.8344x over previous
import functools

import jax
import jax.numpy as jnp
from jax.experimental import pallas as pl

RADIUS = 0.2
DECAY_RADIUS = 0.4


def _prologue_kernel(pts_ref, wct_ref, bc_ref, wst_ref, bs_ref, ctr_ref, dir_ref):
    pts = pts_ref[...]
    ctr_ref[...] = (
        jnp.dot(pts, wct_ref[...], preferred_element_type=jnp.float32) + bc_ref[...]
    )
    dir_ref[...] = (
        jnp.dot(pts, wst_ref[...], preferred_element_type=jnp.float32) + bs_ref[...]
    ).astype(jnp.bfloat16)


def _agg_kernel(xyzr_ref, xyzbr_ref, xyzt_ref, xyzbt_ref, dir_ref, out_ref, *, ti, tj, d):
    n = xyzt_ref.shape[-1]
    r2c = RADIUS * RADIUS
    dr2 = DECAY_RADIUS * DECAY_RADIUS

    mm = jnp.dot(xyzbr_ref[0], xyzbt_ref[0], preferred_element_type=jnp.float32)
    xr = xyzr_ref[0]
    rn = (xr[:, 0:1] * xr[:, 0:1] + xr[:, 1:2] * xr[:, 1:2]) + xr[:, 2:3] * xr[:, 2:3]
    cx = xyzt_ref[0, 0:1, :]
    cy = xyzt_ref[0, 1:2, :]
    cz = xyzt_ref[0, 2:3, :]
    cn = (cx * cx + cy * cy) + cz * cz
    dist = -2.0 * mm
    dist = dist + rn
    dist = dist + cn
    m = jnp.min(dist, axis=1, keepdims=True)
    iota = jax.lax.broadcasted_iota(jnp.int32, (ti, n), 1)
    idx = jnp.min(jnp.where(dist == m, iota, n), axis=1, keepdims=True)
    onehot = (iota == idx).astype(jnp.float32)
    ctr = jax.lax.dot_general(
        onehot, xyzt_ref[0], (((1,), (1,)), ((), ())),
        preferred_element_type=jnp.float32, precision=jax.lax.Precision.HIGHEST,
    )
    xix = ctr[:, 0:1]
    xiy = ctr[:, 1:2]
    xiz = ctr[:, 2:3]

    acc = jnp.zeros((ti, d), jnp.float32)
    z = jnp.zeros((ti, 1), jnp.float32)
    for j0 in range(0, n, tj):
        xjx = xyzt_ref[0, 0:1, j0 : j0 + tj]
        xjy = xyzt_ref[0, 1:2, j0 : j0 + tj]
        xjz = xyzt_ref[0, 2:3, j0 : j0 + tj]
        dx = xjx - xix
        dy = xjy - xiy
        dz = xjz - xiz
        r2 = dx * dx + dy * dy + dz * dz
        rn_ = jnp.sqrt(r2)
        inv = 1.0 / (rn_ + 1e-8)
        dw = 1.0 - (r2 - r2c) / (dr2 - r2c)
        dw = jnp.maximum(dw, 0.0)
        z = z + jnp.sum(dw, axis=1, keepdims=True)
        q = dw * (inv * inv)
        dxb = dx.astype(jnp.bfloat16).astype(jnp.float32)
        dyb = dy.astype(jnp.bfloat16).astype(jnp.float32)
        dzb = dz.astype(jnp.bfloat16).astype(jnp.float32)
        for a2, c in enumerate((dzb, dyb, dxb)):
            s = (c * c) * q
            wp = jnp.where(c > 0.0, s, 0.0)
            wm = s - wp
            for a, w in ((2 * a2, wp), (2 * a2 + 1, wm)):
                acc = acc + jnp.dot(
                    w.astype(jnp.bfloat16),
                    dir_ref[0, j0 : j0 + tj, a * d : (a + 1) * d],
                    preferred_element_type=jnp.float32,
                )
    out_ref[0] = acc / z


def _stats_kernel(x_ref, s_ref):
    x = x_ref[...]
    part = jnp.concatenate(
        [jnp.sum(x, axis=0, keepdims=True), jnp.sum(x * x, axis=0, keepdims=True)], 0
    )

    @pl.when(pl.program_id(0) == 0)
    def _():
        s_ref[...] = part

    @pl.when(pl.program_id(0) != 0)
    def _():
        s_ref[...] += part


def _mid_kernel(
    agg_ref, s1_ref, ctr_ref, g1_ref, b1_ref, w2t_ref, b2_ref, outp_ref, s2_ref, *, nt
):
    s = s1_ref[...]
    m1 = s[0:1] * (1.0 / nt)
    v1 = s[1:2] * (1.0 / nt) - m1 * m1
    dn = (agg_ref[...] - m1) / jnp.sqrt(v1 + 1e-5) * g1_ref[...] + b1_ref[...]
    dn = jnp.maximum(dn, 0.0)
    outp = (
        ctr_ref[...]
        + jnp.dot(dn.astype(jnp.bfloat16), w2t_ref[...],
                  preferred_element_type=jnp.float32)
        + b2_ref[...]
    )
    outp_ref[...] = outp
    part = jnp.concatenate(
        [jnp.sum(outp, axis=0, keepdims=True),
         jnp.sum(outp * outp, axis=0, keepdims=True)], 0
    )

    @pl.when(pl.program_id(0) == 0)
    def _():
        s2_ref[...] = part

    @pl.when(pl.program_id(0) != 0)
    def _():
        s2_ref[...] += part


def _final_kernel(outp_ref, s2_ref, g2_ref, bt2_ref, out_ref, *, nt):
    s = s2_ref[...]
    m2 = s[0:1] * (1.0 / nt)
    v2 = s[1:2] * (1.0 / nt) - m2 * m2
    o = (outp_ref[...] - m2) / jnp.sqrt(v2 + 1e-5) * g2_ref[...] + bt2_ref[...]
    out_ref[...] = jnp.maximum(o, 0.0)


def kernel(
    xyz,
    points,
    W_center,
    b_center,
    W_dir,
    b_dir,
    gamma1,
    beta1,
    W_dir2,
    b_dir2,
    gamma2,
    beta2,
):
    B, N, _ = xyz.shape
    C_IN = points.shape[-1]
    C_OUT = W_center.shape[0]
    D6 = W_dir.shape[0]
    D = D6 // 6

    pts_b = points.reshape(B * N, C_IN).astype(jnp.bfloat16)
    wst = W_dir.reshape(D, 6, C_IN).transpose(1, 0, 2).reshape(D6, C_IN).T
    bst = b_dir.reshape(D, 6).T.reshape(1, D6)

    TR = min(2048, B * N)
    center, dir_stack = pl.pallas_call(
        _prologue_kernel,
        grid=(B * N // TR,),
        in_specs=[
            pl.BlockSpec((TR, C_IN), lambda i: (i, 0)),
            pl.BlockSpec((C_IN, C_OUT), lambda i: (0, 0)),
            pl.BlockSpec((1, C_OUT), lambda i: (0, 0)),
            pl.BlockSpec((C_IN, D6), lambda i: (0, 0)),
            pl.BlockSpec((1, D6), lambda i: (0, 0)),
        ],
        out_specs=[
            pl.BlockSpec((TR, C_OUT), lambda i: (i, 0)),
            pl.BlockSpec((TR, D6), lambda i: (i, 0)),
        ],
        out_shape=[
            jax.ShapeDtypeStruct((B * N, C_OUT), jnp.float32),
            jax.ShapeDtypeStruct((B * N, D6), jnp.bfloat16),
        ],
    )(pts_b, W_center.T.astype(jnp.bfloat16), b_center.reshape(1, C_OUT),
      wst.astype(jnp.bfloat16), bst)

    dir_b = dir_stack.reshape(B, N, D6)
    xyz_t = xyz.transpose(0, 2, 1)
    xyz_b16 = xyz.astype(jnp.bfloat16)
    xyz_t_b16 = xyz_t.astype(jnp.bfloat16)

    TI = min(256, N)
    TJ = min(512, N)
    agg = pl.pallas_call(
        functools.partial(_agg_kernel, ti=TI, tj=TJ, d=D),
        grid=(B, N // TI),
        in_specs=[
            pl.BlockSpec((1, TI, 3), lambda b, i: (b, i, 0)),
            pl.BlockSpec((1, TI, 3), lambda b, i: (b, i, 0)),
            pl.BlockSpec((1, 3, N), lambda b, i: (b, 0, 0)),
            pl.BlockSpec((1, 3, N), lambda b, i: (b, 0, 0)),
            pl.BlockSpec((1, N, D6), lambda b, i: (b, 0, 0)),
        ],
        out_specs=pl.BlockSpec((1, TI, D), lambda b, i: (b, i, 0)),
        out_shape=jax.ShapeDtypeStruct((B, N, D), jnp.float32),
    )(xyz, xyz_b16, xyz_t, xyz_t_b16, dir_b)

    agg_flat = agg.reshape(B * N, D)
    NT = B * N
    TE = min(2048, NT)
    GE = NT // TE
    s1 = pl.pallas_call(
        _stats_kernel,
        grid=(GE,),
        in_specs=[pl.BlockSpec((TE, D), lambda i: (i, 0))],
        out_specs=pl.BlockSpec((2, D), lambda i: (0, 0)),
        out_shape=jax.ShapeDtypeStruct((2, D), jnp.float32),
    )(agg_flat)

    outp, s2 = pl.pallas_call(
        functools.partial(_mid_kernel, nt=NT),
        grid=(GE,),
        in_specs=[
            pl.BlockSpec((TE, D), lambda i: (i, 0)),
            pl.BlockSpec((2, D), lambda i: (0, 0)),
            pl.BlockSpec((TE, C_OUT), lambda i: (i, 0)),
            pl.BlockSpec((1, D), lambda i: (0, 0)),
            pl.BlockSpec((1, D), lambda i: (0, 0)),
            pl.BlockSpec((D, C_OUT), lambda i: (0, 0)),
            pl.BlockSpec((1, C_OUT), lambda i: (0, 0)),
        ],
        out_specs=[
            pl.BlockSpec((TE, C_OUT), lambda i: (i, 0)),
            pl.BlockSpec((2, C_OUT), lambda i: (0, 0)),
        ],
        out_shape=[
            jax.ShapeDtypeStruct((NT, C_OUT), jnp.float32),
            jax.ShapeDtypeStruct((2, C_OUT), jnp.float32),
        ],
    )(agg_flat, s1, center, gamma1.reshape(1, D), beta1.reshape(1, D),
      W_dir2.T.astype(jnp.bfloat16), b_dir2.reshape(1, C_OUT))

    out = pl.pallas_call(
        functools.partial(_final_kernel, nt=NT),
        grid=(GE,),
        in_specs=[
            pl.BlockSpec((TE, C_OUT), lambda i: (i, 0)),
            pl.BlockSpec((2, C_OUT), lambda i: (0, 0)),
            pl.BlockSpec((1, C_OUT), lambda i: (0, 0)),
            pl.BlockSpec((1, C_OUT), lambda i: (0, 0)),
        ],
        out_specs=pl.BlockSpec((TE, C_OUT), lambda i: (i, 0)),
        out_shape=jax.ShapeDtypeStruct((NT, C_OUT), jnp.float32),
    )(outp, s2, gamma2.reshape(1, C_OUT), beta2.reshape(1, C_OUT))
    return out.reshape(B, N, C_OUT)


# speedup vs baseline: 83.4892x; 1.5865x over previous
import functools

import jax
import jax.numpy as jnp
from jax.experimental import pallas as pl

RADIUS = 0.2
DECAY_RADIUS = 0.4


def _prologue_kernel(pts_ref, wct_ref, bc_ref, wst_ref, bs_ref, ctr_ref, dir_ref):
    pts = pts_ref[...]
    ctr_ref[...] = (
        jnp.dot(pts, wct_ref[...], preferred_element_type=jnp.float32) + bc_ref[...]
    )
    dir_ref[...] = (
        jnp.dot(pts, wst_ref[...], preferred_element_type=jnp.float32) + bs_ref[...]
    ).astype(jnp.bfloat16)


def _agg_kernel(xyzr_ref, xyzbr_ref, xyzt_ref, xyzbt_ref, dir_ref, out_ref, *, ti, tj, d):
    n = xyzt_ref.shape[-1]
    r2c = RADIUS * RADIUS
    dr2 = DECAY_RADIUS * DECAY_RADIUS

    mm = jnp.dot(xyzbr_ref[0], xyzbt_ref[0], preferred_element_type=jnp.float32)
    xr = xyzr_ref[0]
    rn = (xr[:, 0:1] * xr[:, 0:1] + xr[:, 1:2] * xr[:, 1:2]) + xr[:, 2:3] * xr[:, 2:3]
    cx = xyzt_ref[0, 0:1, :]
    cy = xyzt_ref[0, 1:2, :]
    cz = xyzt_ref[0, 2:3, :]
    cn = (cx * cx + cy * cy) + cz * cz
    dist = -2.0 * mm
    dist = dist + rn
    dist = dist + cn
    m = jnp.min(dist, axis=1, keepdims=True)
    iota = jax.lax.broadcasted_iota(jnp.int32, (ti, n), 1)
    idx = jnp.min(jnp.where(dist == m, iota, n), axis=1, keepdims=True)
    sel = iota == idx
    xix = jnp.sum(jnp.where(sel, cx, 0.0), axis=1, keepdims=True)
    xiy = jnp.sum(jnp.where(sel, cy, 0.0), axis=1, keepdims=True)
    xiz = jnp.sum(jnp.where(sel, cz, 0.0), axis=1, keepdims=True)
    acc = jnp.zeros((ti, d), jnp.float32)
    z = jnp.zeros((ti, 1), jnp.float32)
    for j0 in range(0, n, tj):
        xjx = xyzt_ref[0, 0:1, j0 : j0 + tj]
        xjy = xyzt_ref[0, 1:2, j0 : j0 + tj]
        xjz = xyzt_ref[0, 2:3, j0 : j0 + tj]
        dx = xjx - xix
        dy = xjy - xiy
        dz = xjz - xiz
        r2 = dx * dx + dy * dy + dz * dz
        rn_ = jnp.sqrt(r2)
        inv = 1.0 / (rn_ + 1e-8)
        dw = 1.0 - (r2 - r2c) / (dr2 - r2c)
        dw = jnp.maximum(dw, 0.0)
        z = z + jnp.sum(dw, axis=1, keepdims=True)
        q = dw * (inv * inv)
        dxb = dx.astype(jnp.bfloat16).astype(jnp.float32)
        dyb = dy.astype(jnp.bfloat16).astype(jnp.float32)
        dzb = dz.astype(jnp.bfloat16).astype(jnp.float32)
        for a2, c in enumerate((dzb, dyb, dxb)):
            s = (c * c) * q
            wp = jnp.where(c > 0.0, s, 0.0)
            wm = s - wp
            for a, w in ((2 * a2, wp), (2 * a2 + 1, wm)):
                acc = acc + jnp.dot(
                    w.astype(jnp.bfloat16),
                    dir_ref[0, j0 : j0 + tj, a * d : (a + 1) * d],
                    preferred_element_type=jnp.float32,
                )
    out_ref[0] = acc / z


def _stats_kernel(x_ref, s_ref):
    x = x_ref[...]
    part = jnp.concatenate(
        [jnp.sum(x, axis=0, keepdims=True), jnp.sum(x * x, axis=0, keepdims=True)], 0
    )

    @pl.when(pl.program_id(0) == 0)
    def _():
        s_ref[...] = part

    @pl.when(pl.program_id(0) != 0)
    def _():
        s_ref[...] += part


def _mid_kernel(
    agg_ref, s1_ref, ctr_ref, g1_ref, b1_ref, w2t_ref, b2_ref, outp_ref, s2_ref, *, nt
):
    s = s1_ref[...]
    m1 = s[0:1] * (1.0 / nt)
    v1 = s[1:2] * (1.0 / nt) - m1 * m1
    dn = (agg_ref[...] - m1) / jnp.sqrt(v1 + 1e-5) * g1_ref[...] + b1_ref[...]
    dn = jnp.maximum(dn, 0.0)
    outp = (
        ctr_ref[...]
        + jnp.dot(dn.astype(jnp.bfloat16), w2t_ref[...],
                  preferred_element_type=jnp.float32)
        + b2_ref[...]
    )
    outp_ref[...] = outp
    part = jnp.concatenate(
        [jnp.sum(outp, axis=0, keepdims=True),
         jnp.sum(outp * outp, axis=0, keepdims=True)], 0
    )

    @pl.when(pl.program_id(0) == 0)
    def _():
        s2_ref[...] = part

    @pl.when(pl.program_id(0) != 0)
    def _():
        s2_ref[...] += part


def _final_kernel(outp_ref, s2_ref, g2_ref, bt2_ref, out_ref, *, nt):
    s = s2_ref[...]
    m2 = s[0:1] * (1.0 / nt)
    v2 = s[1:2] * (1.0 / nt) - m2 * m2
    o = (outp_ref[...] - m2) / jnp.sqrt(v2 + 1e-5) * g2_ref[...] + bt2_ref[...]
    out_ref[...] = jnp.maximum(o, 0.0)


def kernel(
    xyz,
    points,
    W_center,
    b_center,
    W_dir,
    b_dir,
    gamma1,
    beta1,
    W_dir2,
    b_dir2,
    gamma2,
    beta2,
):
    B, N, _ = xyz.shape
    C_IN = points.shape[-1]
    C_OUT = W_center.shape[0]
    D6 = W_dir.shape[0]
    D = D6 // 6

    pts_b = points.reshape(B * N, C_IN).astype(jnp.bfloat16)
    wst = W_dir.reshape(D, 6, C_IN).transpose(1, 0, 2).reshape(D6, C_IN).T
    bst = b_dir.reshape(D, 6).T.reshape(1, D6)

    TR = min(2048, B * N)
    center, dir_stack = pl.pallas_call(
        _prologue_kernel,
        grid=(B * N // TR,),
        in_specs=[
            pl.BlockSpec((TR, C_IN), lambda i: (i, 0)),
            pl.BlockSpec((C_IN, C_OUT), lambda i: (0, 0)),
            pl.BlockSpec((1, C_OUT), lambda i: (0, 0)),
            pl.BlockSpec((C_IN, D6), lambda i: (0, 0)),
            pl.BlockSpec((1, D6), lambda i: (0, 0)),
        ],
        out_specs=[
            pl.BlockSpec((TR, C_OUT), lambda i: (i, 0)),
            pl.BlockSpec((TR, D6), lambda i: (i, 0)),
        ],
        out_shape=[
            jax.ShapeDtypeStruct((B * N, C_OUT), jnp.float32),
            jax.ShapeDtypeStruct((B * N, D6), jnp.bfloat16),
        ],
    )(pts_b, W_center.T.astype(jnp.bfloat16), b_center.reshape(1, C_OUT),
      wst.astype(jnp.bfloat16), bst)

    dir_b = dir_stack.reshape(B, N, D6)
    xyz_t = xyz.transpose(0, 2, 1)
    xyz_b16 = xyz.astype(jnp.bfloat16)
    xyz_t_b16 = xyz_t.astype(jnp.bfloat16)

    TI = min(256, N)
    TJ = min(512, N)
    agg = pl.pallas_call(
        functools.partial(_agg_kernel, ti=TI, tj=TJ, d=D),
        grid=(B, N // TI),
        in_specs=[
            pl.BlockSpec((1, TI, 3), lambda b, i: (b, i, 0)),
            pl.BlockSpec((1, TI, 3), lambda b, i: (b, i, 0)),
            pl.BlockSpec((1, 3, N), lambda b, i: (b, 0, 0)),
            pl.BlockSpec((1, 3, N), lambda b, i: (b, 0, 0)),
            pl.BlockSpec((1, N, D6), lambda b, i: (b, 0, 0)),
        ],
        out_specs=pl.BlockSpec((1, TI, D), lambda b, i: (b, i, 0)),
        out_shape=jax.ShapeDtypeStruct((B, N, D), jnp.float32),
    )(xyz, xyz_b16, xyz_t, xyz_t_b16, dir_b)

    agg_flat = agg.reshape(B * N, D)
    NT = B * N
    TE = min(2048, NT)
    GE = NT // TE
    s1 = pl.pallas_call(
        _stats_kernel,
        grid=(GE,),
        in_specs=[pl.BlockSpec((TE, D), lambda i: (i, 0))],
        out_specs=pl.BlockSpec((2, D), lambda i: (0, 0)),
        out_shape=jax.ShapeDtypeStruct((2, D), jnp.float32),
    )(agg_flat)

    outp, s2 = pl.pallas_call(
        functools.partial(_mid_kernel, nt=NT),
        grid=(GE,),
        in_specs=[
            pl.BlockSpec((TE, D), lambda i: (i, 0)),
            pl.BlockSpec((2, D), lambda i: (0, 0)),
            pl.BlockSpec((TE, C_OUT), lambda i: (i, 0)),
            pl.BlockSpec((1, D), lambda i: (0, 0)),
            pl.BlockSpec((1, D), lambda i: (0, 0)),
            pl.BlockSpec((D, C_OUT), lambda i: (0, 0)),
            pl.BlockSpec((1, C_OUT), lambda i: (0, 0)),
        ],
        out_specs=[
            pl.BlockSpec((TE, C_OUT), lambda i: (i, 0)),
            pl.BlockSpec((2, C_OUT), lambda i: (0, 0)),
        ],
        out_shape=[
            jax.ShapeDtypeStruct((NT, C_OUT), jnp.float32),
            jax.ShapeDtypeStruct((2, C_OUT), jnp.float32),
        ],
    )(agg_flat, s1, center, gamma1.reshape(1, D), beta1.reshape(1, D),
      W_dir2.T.astype(jnp.bfloat16), b_dir2.reshape(1, C_OUT))

    out = pl.pallas_call(
        functools.partial(_final_kernel, nt=NT),
        grid=(GE,),
        in_specs=[
            pl.BlockSpec((TE, C_OUT), lambda i: (i, 0)),
            pl.BlockSpec((2, C_OUT), lambda i: (0, 0)),
            pl.BlockSpec((1, C_OUT), lambda i: (0, 0)),
            pl.BlockSpec((1, C_OUT), lambda i: (0, 0)),
        ],
        out_specs=pl.BlockSpec((TE, C_OUT), lambda i: (i, 0)),
        out_shape=jax.ShapeDtypeStruct((NT, C_OUT), jnp.float32),
    )(outp, s2, gamma2.reshape(1, C_OUT), beta2.reshape(1, C_OUT))
    return out.reshape(B, N, C_OUT)
